# baseline (speedup 1.0000x reference)
import functools

import jax
import jax.numpy as jnp
from jax import lax
from jax.experimental import pallas as pl
from jax.experimental.pallas import tpu as pltpu
from jax.experimental.pallas import tpu_sc as plsc

NUM_CLS = 32
D_MODEL = 256
BATCH = 4096
HIST = 50

NC, NS = 2, 16
NW = NC * NS
ROWS = BATCH * HIST
R_PER_W = ROWS // NW
K_REP = 8
CHUNK = 128
NCHUNK = R_PER_W // CHUNK
PAIRS = NCHUNK // 2

FB = 512
NFB = BATCH // FB
NBLK = HIST * NFB


@functools.partial(
    pl.kernel,
    out_type=jax.ShapeDtypeStruct((ROWS, D_MODEL), jnp.float32),
    mesh=plsc.VectorSubcoreMesh(core_axis_name="c", subcore_axis_name="s"),
    scratch_types=[
        pltpu.VMEM((NCHUNK, CHUNK), jnp.int32),
        pltpu.VMEM((CHUNK, D_MODEL), jnp.float32),
        pltpu.VMEM((CHUNK, D_MODEL), jnp.float32),
        pltpu.SemaphoreType.DMA,
        pltpu.SemaphoreType.DMA,
        pltpu.SemaphoreType.DMA,
        pltpu.SemaphoreType.DMA,
    ],
)
def _embed_sc(table_hbm, idx_hbm, out_hbm, idx_v, buf_a, buf_b,
              gsem_a, gsem_b, wsem_a, wsem_b):
    wid = lax.axis_index("s") * NC + lax.axis_index("c")
    base = wid * R_PER_W

    pltpu.sync_copy(idx_hbm.at[wid], idx_v)

    def gather(c, buf, sem):
        pltpu.async_copy(table_hbm.at[idx_v.at[c]], buf, sem)

    def wait_gather(c, buf, sem):
        pltpu.make_async_copy(table_hbm.at[idx_v.at[c]], buf, sem).wait()

    def write(c, buf, sem):
        pltpu.async_copy(buf, out_hbm.at[pl.ds(base + c * CHUNK, CHUNK)], sem)

    def wait_write(c, buf, sem):
        pltpu.make_async_copy(
            buf, out_hbm.at[pl.ds(base + c * CHUNK, CHUNK)], sem).wait()

    gather(0, buf_a, gsem_a)

    def pair(i):
        c0 = i * 2
        gather(c0 + 1, buf_b, gsem_b)
        wait_gather(c0, buf_a, gsem_a)
        write(c0, buf_a, wsem_a)
        @pl.when(i < PAIRS - 1)
        def _():
            wait_write(c0, buf_a, wsem_a)
            gather(c0 + 2, buf_a, gsem_a)
        wait_gather(c0 + 1, buf_b, gsem_b)
        write(c0 + 1, buf_b, wsem_b)
        @pl.when(i < PAIRS - 1)
        def _():
            wait_write(c0 + 1, buf_b, wsem_b)

    pl.loop(0, PAIRS)(pair)
    wait_write(NCHUNK - 2, buf_a, wsem_a)
    wait_write(NCHUNK - 1, buf_b, wsem_b)


def _finish_body(rows_hbm, out_hbm, sem):
    copies = [
        pltpu.make_async_copy(
            rows_hbm.at[pl.ds(h, 1)], out_hbm.at[pl.ds(h, 1)], sem)
        for h in range(HIST)
    ]
    for c in copies:
        c.start()
    for c in copies:
        c.wait()


_finish = pl.pallas_call(
    _finish_body,
    in_specs=[pl.BlockSpec(memory_space=pl.ANY)],
    out_specs=pl.BlockSpec(memory_space=pl.ANY),
    out_shape=jax.ShapeDtypeStruct((HIST, BATCH, D_MODEL), jnp.float32),
    scratch_shapes=[pltpu.SemaphoreType.DMA],
)


def kernel(data, table):
    flat = data.T.reshape(-1)
    i = jnp.arange(ROWS, dtype=jnp.int32)
    offs = (i // R_PER_W) * K_REP + (i % K_REP)
    idx = (flat + NUM_CLS * offs).reshape(NW, NCHUNK, CHUNK)
    rep = jnp.tile(table, (NW * K_REP, 1))
    rows = _embed_sc(rep, idx).reshape(HIST, BATCH, D_MODEL)
    return _finish(rows).transpose(1, 0, 2)


# speedup vs baseline: 12.4601x; 12.4601x over previous
import functools

import jax
import jax.numpy as jnp
from jax import lax
from jax.experimental import pallas as pl
from jax.experimental.pallas import tpu as pltpu
from jax.experimental.pallas import tpu_sc as plsc

NUM_CLS = 32
D_MODEL = 256
BATCH = 4096
HIST = 50

NC, NS = 2, 16
NW = NC * NS
ROWS = BATCH * HIST
R_PER_W = ROWS // NW
K_REP = 8
CHUNK = 128
NCHUNK = R_PER_W // CHUNK
PAIRS = NCHUNK // 2

FB = 1024
NFB = BATCH // FB
NBLK = HIST * NFB


@functools.partial(
    pl.kernel,
    out_type=jax.ShapeDtypeStruct((ROWS, D_MODEL), jnp.float32),
    mesh=plsc.VectorSubcoreMesh(core_axis_name="c", subcore_axis_name="s"),
    scratch_types=[
        pltpu.VMEM((NCHUNK, CHUNK), jnp.int32),
        pltpu.VMEM((CHUNK, D_MODEL), jnp.float32),
        pltpu.VMEM((CHUNK, D_MODEL), jnp.float32),
        pltpu.SemaphoreType.DMA,
        pltpu.SemaphoreType.DMA,
        pltpu.SemaphoreType.DMA,
        pltpu.SemaphoreType.DMA,
    ],
)
def _embed_sc(table_hbm, idx_hbm, out_hbm, idx_v, buf_a, buf_b,
              gsem_a, gsem_b, wsem_a, wsem_b):
    wid = lax.axis_index("s") * NC + lax.axis_index("c")
    base = wid * R_PER_W

    pltpu.sync_copy(idx_hbm.at[wid], idx_v)

    def gather(c, buf, sem):
        pltpu.async_copy(table_hbm.at[idx_v.at[c]], buf, sem)

    def wait_gather(c, buf, sem):
        pltpu.make_async_copy(table_hbm.at[idx_v.at[c]], buf, sem).wait()

    def write(c, buf, sem):
        pltpu.async_copy(buf, out_hbm.at[pl.ds(base + c * CHUNK, CHUNK)], sem)

    def wait_write(c, buf, sem):
        pltpu.make_async_copy(
            buf, out_hbm.at[pl.ds(base + c * CHUNK, CHUNK)], sem).wait()

    gather(0, buf_a, gsem_a)

    def pair(i):
        c0 = i * 2
        gather(c0 + 1, buf_b, gsem_b)
        wait_gather(c0, buf_a, gsem_a)
        write(c0, buf_a, wsem_a)
        @pl.when(i < PAIRS - 1)
        def _():
            wait_write(c0, buf_a, wsem_a)
            gather(c0 + 2, buf_a, gsem_a)
        wait_gather(c0 + 1, buf_b, gsem_b)
        write(c0 + 1, buf_b, wsem_b)
        @pl.when(i < PAIRS - 1)
        def _():
            wait_write(c0 + 1, buf_b, wsem_b)

    pl.loop(0, PAIRS)(pair)
    wait_write(NCHUNK - 2, buf_a, wsem_a)
    wait_write(NCHUNK - 1, buf_b, wsem_b)


def _finish_body(rows_hbm, out_ref, sem):
    b = pl.program_id(0)
    pltpu.async_copy(
        rows_hbm.at[pl.ds(b // NFB, 1), pl.ds((b % NFB) * FB, FB)],
        out_ref, sem).wait()


_finish = pl.pallas_call(
    _finish_body,
    grid=(NBLK,),
    in_specs=[pl.BlockSpec(memory_space=pl.ANY)],
    out_specs=pl.BlockSpec((1, FB, D_MODEL), lambda b: (b // NFB, b % NFB, 0)),
    out_shape=jax.ShapeDtypeStruct((HIST, BATCH, D_MODEL), jnp.float32),
    scratch_shapes=[pltpu.SemaphoreType.DMA],
)


def kernel(data, table):
    flat = data.T.reshape(-1)
    i = jnp.arange(ROWS, dtype=jnp.int32)
    offs = (i // R_PER_W) * K_REP + (i % K_REP)
    idx = (flat + NUM_CLS * offs).reshape(NW, NCHUNK, CHUNK)
    rep = jnp.tile(table, (NW * K_REP, 1))
    rows = _embed_sc(rep, idx).reshape(HIST, BATCH, D_MODEL)
    return _finish(rows).transpose(1, 0, 2)


# speedup vs baseline: 20.0031x; 1.6054x over previous
import functools

import jax
import jax.numpy as jnp
from jax import lax
from jax.experimental import pallas as pl
from jax.experimental.pallas import tpu as pltpu
from jax.experimental.pallas import tpu_sc as plsc

NUM_CLS = 32
D_MODEL = 256
BATCH = 4096
HIST = 50

NC, NS = 2, 16
NW = NC * NS
ROWS = BATCH * HIST
R_PER_W = ROWS // NW
K_REP = 8
CHUNK = 128
NCHUNK = R_PER_W // CHUNK
PAIRS = NCHUNK // 2

FB = 1024
NFB = BATCH // FB
NBLK = HIST * NFB


@functools.partial(
    pl.kernel,
    out_type=jax.ShapeDtypeStruct((ROWS, D_MODEL), jnp.float32),
    mesh=plsc.VectorSubcoreMesh(core_axis_name="c", subcore_axis_name="s"),
    scratch_types=[
        pltpu.VMEM((NCHUNK, CHUNK), jnp.int32),
        pltpu.VMEM((CHUNK, D_MODEL), jnp.float32),
        pltpu.VMEM((CHUNK, D_MODEL), jnp.float32),
        pltpu.SemaphoreType.DMA,
        pltpu.SemaphoreType.DMA,
        pltpu.SemaphoreType.DMA,
        pltpu.SemaphoreType.DMA,
    ],
)
def _embed_sc(table_hbm, idx_hbm, out_hbm, idx_v, buf_a, buf_b,
              gsem_a, gsem_b, wsem_a, wsem_b):
    wid = lax.axis_index("s") * NC + lax.axis_index("c")
    base = wid * R_PER_W

    pltpu.sync_copy(idx_hbm.at[wid], idx_v)

    def gather(c, buf, sem):
        pltpu.async_copy(table_hbm.at[idx_v.at[c]], buf, sem)

    def wait_gather(c, buf, sem):
        pltpu.make_async_copy(table_hbm.at[idx_v.at[c]], buf, sem).wait()

    def write(c, buf, sem):
        pltpu.async_copy(buf, out_hbm.at[pl.ds(base + c * CHUNK, CHUNK)], sem)

    def wait_write(c, buf, sem):
        pltpu.make_async_copy(
            buf, out_hbm.at[pl.ds(base + c * CHUNK, CHUNK)], sem).wait()

    gather(0, buf_a, gsem_a)

    def pair(i):
        c0 = i * 2
        gather(c0 + 1, buf_b, gsem_b)
        wait_gather(c0, buf_a, gsem_a)
        write(c0, buf_a, wsem_a)
        @pl.when(i < PAIRS - 1)
        def _():
            wait_write(c0, buf_a, wsem_a)
            gather(c0 + 2, buf_a, gsem_a)
        wait_gather(c0 + 1, buf_b, gsem_b)
        write(c0 + 1, buf_b, wsem_b)
        @pl.when(i < PAIRS - 1)
        def _():
            wait_write(c0 + 1, buf_b, wsem_b)

    pl.loop(0, PAIRS)(pair)
    wait_write(NCHUNK - 2, buf_a, wsem_a)
    wait_write(NCHUNK - 1, buf_b, wsem_b)


NLEAD = 2
NBUF = NLEAD + 1


def _finish_body(rows_hbm, out_ref, in_a, in_b, in_c, sem_a, sem_b, sem_c):
    b = pl.program_id(0)
    ins = (in_a, in_b, in_c)
    sems = (sem_a, sem_b, sem_c)

    def load(blk, par):
        return pltpu.make_async_copy(
            rows_hbm.at[pl.ds(blk * FB, FB)], ins[par], sems[par])

    @pl.when(b == 0)
    def _():
        for k in range(NLEAD):
            load(k, k).start()

    @pl.when(b + NLEAD < NBLK)
    def _():
        for par in range(NBUF):
            @pl.when(lax.rem(b + NLEAD, NBUF) == par)
            def _(par=par):
                load(b + NLEAD, par).start()

    for par in range(NBUF):
        @pl.when(lax.rem(b, NBUF) == par)
        def _(par=par):
            load(b, par).wait()
            out_ref[...] = ins[par][...].reshape(1, FB, D_MODEL)


_finish = pl.pallas_call(
    _finish_body,
    grid=(NBLK,),
    in_specs=[pl.BlockSpec(memory_space=pl.ANY)],
    out_specs=pl.BlockSpec((1, FB, D_MODEL), lambda b: (b // NFB, b % NFB, 0)),
    out_shape=jax.ShapeDtypeStruct((HIST, BATCH, D_MODEL), jnp.float32),
    scratch_shapes=[
        pltpu.VMEM((FB, D_MODEL), jnp.float32),
        pltpu.VMEM((FB, D_MODEL), jnp.float32),
        pltpu.VMEM((FB, D_MODEL), jnp.float32),
        pltpu.SemaphoreType.DMA,
        pltpu.SemaphoreType.DMA,
        pltpu.SemaphoreType.DMA,
    ],
)


def kernel(data, table):
    flat = data.T.reshape(-1)
    i = jnp.arange(ROWS, dtype=jnp.int32)
    offs = (i // R_PER_W) * K_REP + (i % K_REP)
    idx = (flat + NUM_CLS * offs).reshape(NW, NCHUNK, CHUNK)
    rep = jnp.tile(table, (NW * K_REP, 1))
    rows = _embed_sc(rep, idx)
    return _finish(rows).transpose(1, 0, 2)
